# baseline (speedup 1.0000x reference)
import functools

import jax
import jax.numpy as jnp
from jax import lax
from jax.experimental import pallas as pl
from jax.experimental.pallas import tpu as pltpu
from jax.experimental.pallas import tpu_sc as plsc

_K = 16
_TILE_M = 256
_F = 16


def _knn_kernel(q_ref, pt_ref, idx_ref, d_ref):
    pt = pt_ref[0]
    n = pt.shape[1]
    q = q_ref[0]
    qx = q[:, 0:1]
    qy = q[:, 1:2]
    qz = q[:, 2:3]
    q2 = qx * qx + qy * qy + qz * qz
    px = pt[0:1, :]
    py = pt[1:2, :]
    pz = pt[2:3, :]
    p2 = px * px + py * py + pz * pz
    qp = jax.lax.dot_general(
        q.astype(jnp.bfloat16), pt.astype(jnp.bfloat16),
        (((1,), (0,)), ((), ())), preferred_element_type=jnp.float32)
    d_ref[...] = (q2 + p2) - 2.0 * qp

    lw = n // _F
    iota_l = lax.broadcasted_iota(jnp.int32, (1, lw), 1)
    inf = jnp.float32(jnp.inf)
    c1 = d_ref[:, 0:lw]
    i1 = iota_l
    c2 = jnp.full((1, lw), inf, jnp.float32)
    i2 = jnp.full((1, lw), n, jnp.int32)
    c3, i3 = c2, i2
    for f in range(1, _F):
        df = d_ref[:, f * lw:(f + 1) * lw]
        ci = iota_l + f * lw
        lt1 = df < c1
        nc1 = jnp.where(lt1, df, c1)
        ni1 = jnp.where(lt1, ci, i1)
        sv = jnp.where(lt1, c1, df)
        si = jnp.where(lt1, i1, ci)
        lt2 = sv < c2
        nc2 = jnp.where(lt2, sv, c2)
        ni2 = jnp.where(lt2, si, i2)
        tv = jnp.where(lt2, c2, sv)
        ti = jnp.where(lt2, i2, si)
        lt3 = tv < c3
        c3 = jnp.where(lt3, tv, c3)
        i3 = jnp.where(lt3, ti, i3)
        c1, i1, c2, i2 = nc1, ni1, nc2, ni2

    def _cx(a, b):
        w = a[0] <= b[0]
        return ((jnp.where(w, a[0], b[0]), jnp.where(w, a[1], b[1])),
                (jnp.where(w, b[0], a[0]), jnp.where(w, b[1], a[1])))

    def _cxmin(a, b):
        w = a[0] <= b[0]
        return (jnp.where(w, a[0], b[0]), jnp.where(w, a[1], b[1]))

    def _merge33(a, b):
        x1, y1 = _cx(a[0], b[0])
        x2, y2 = _cx(a[1], b[1])
        x3 = _cxmin(a[2], b[2])
        p, q = _cx(y1, x2)
        r, s = _cx(q, x3)
        t = _cxmin(s, y2)
        return (x1, p, r, t)

    def _merge44(a, b):
        x1, y1 = _cx(a[0], b[0])
        x2, y2 = _cx(a[1], b[1])
        x3 = _cxmin(a[2], b[2])
        x4 = _cxmin(a[3], b[3])
        p, q = _cx(y1, x2)
        r, s = _cx(q, x3)
        t = _cxmin(s, y2)
        u = _cxmin(t, x4)
        return (x1, p, r, u)

    lw2 = lw // 4
    quads = []
    for k in range(4):
        sl = slice(k * lw2, (k + 1) * lw2)
        quads.append(((c1[:, sl], i1[:, sl]),
                      (c2[:, sl], i2[:, sl]),
                      (c3[:, sl], i3[:, sl])))
    pm = _merge33(quads[0], quads[1])
    qm = _merge33(quads[2], quads[3])
    (c1, i1), (c2, i2), (c3, i3), (c4, i4) = _merge44(pm, qm)

    cols = []
    ms = []
    m16 = None
    i16 = None
    for _ in range(_K):
        m16 = jnp.min(c1, axis=1, keepdims=True)
        i16 = jnp.min(jnp.where(c1 == m16, i1, n), axis=1, keepdims=True)
        cols.append(i16)
        ms.append(m16)
        lm = i1 == i16
        c1 = jnp.where(lm, c2, c1)
        i1 = jnp.where(lm, i2, i1)
        c2 = jnp.where(lm, c3, c2)
        i2 = jnp.where(lm, i3, i2)
        c3 = jnp.where(lm, c4, c3)
        i3 = jnp.where(lm, i4, i3)
        c4 = jnp.where(lm, inf, c4)
        i4 = jnp.where(lm, n, i4)
    idx_ref[0] = jnp.concatenate(cols, axis=1)

    iota_n = lax.broadcasted_iota(jnp.int32, (1, n), 1)
    d = d_ref[...]
    cnt = jnp.sum((d <= m16).astype(jnp.int32), axis=1, keepdims=True)
    ordered = None
    for t in range(1, _K):
        inc = (ms[t - 1] < ms[t]) | ((ms[t - 1] == ms[t]) & (cols[t - 1] < cols[t]))
        ordered = inc if ordered is None else (ordered & inc)
    bad = jnp.any(cnt != _K) | jnp.any(jnp.logical_not(ordered))

    @pl.when(bad)
    def _slow_path():
        scols = []
        for _ in range(_K):
            dd = d_ref[...]
            m = jnp.min(dd, axis=1, keepdims=True)
            ii = jnp.min(jnp.where(dd == m, iota_n, n), axis=1, keepdims=True)
            scols.append(ii)
            d_ref[...] = jnp.where(iota_n == ii, inf, dd)
        idx_ref[0] = jnp.concatenate(scols, axis=1)


@functools.partial(jax.jit, static_argnames=())
def _knn_idx(xyz, xyz_t):
    b, m, _ = xyz.shape
    n = xyz_t.shape[2]
    return pl.pallas_call(
        _knn_kernel,
        grid=(b, m // _TILE_M),
        in_specs=[
            pl.BlockSpec((1, _TILE_M, 3), lambda i, j: (i, j, 0)),
            pl.BlockSpec((1, 3, n), lambda i, j: (i, 0, 0)),
        ],
        out_specs=pl.BlockSpec((1, _TILE_M, _K), lambda i, j: (i, j, 0)),
        out_shape=jax.ShapeDtypeStruct((b, m, _K), jnp.int32),
        scratch_shapes=[pltpu.VMEM((_TILE_M, n), jnp.float32)],
    )(xyz, xyz_t)


_NC = 2
_NS = 16


def _gather_sc(xyz_t, idx):
    b_sz, _, n = xyz_t.shape
    m = idx.shape[1]
    nw = _NC * _NS
    ch = m // nw
    mesh = plsc.VectorSubcoreMesh(core_axis_name="c", subcore_axis_name="s")

    @functools.partial(
        pl.kernel, mesh=mesh,
        compiler_params=pltpu.CompilerParams(needs_layout_passes=False),
        out_type=jax.ShapeDtypeStruct((b_sz * _K * 3 * m,), jnp.float32),
        scratch_types=[
            pltpu.VMEM((3 * n,), jnp.float32),
            pltpu.VMEM((ch * _K,), jnp.int32),
            pltpu.VMEM((_K * 3 * ch,), jnp.float32),
        ],
    )
    def k(xyz_hbm, idx_hbm, out_hbm, tab_v, idx_v, out_v):
        wid = lax.axis_index("s") * _NC + lax.axis_index("c")
        base = wid * ch
        jstride = lax.iota(jnp.int32, _K) * (3 * ch)
        for b in range(b_sz):
            pltpu.sync_copy(xyz_hbm.at[pl.ds(b * 3 * n, 3 * n)], tab_v)
            pltpu.sync_copy(
                idx_hbm.at[pl.ds((b * m + base) * _K, ch * _K)], idx_v)

            def body(q, _):
                iv = idx_v[pl.ds(q * _K, _K)]
                for c in range(3):
                    vals = plsc.load_gather(tab_v, [iv + (c * n)])
                    plsc.store_scatter(out_v, [jstride + (c * ch + q)], vals)
                return 0

            lax.fori_loop(0, ch, body, 0)
            for j in range(_K):
                for c in range(3):
                    pltpu.sync_copy(
                        out_v.at[pl.ds((j * 3 + c) * ch, ch)],
                        out_hbm.at[pl.ds(((b * _K + j) * 3 + c) * m + base, ch)])

    return k(xyz_t.reshape(-1), idx.reshape(-1)).reshape(b_sz, _K, 3, m)


_TILE_T = 4096


def _tail_kernel(g_ref, c_ref, wp_ref, ws_ref, bs_ref, out_ref):
    g = g_ref[0]
    ctr = c_ref[0]
    wp = wp_ref[...]
    ws = ws_ref[...]
    bs = bs_ref[...]
    acc = None
    for j in range(1, _K):
        v = g[j] - ctr
        n2 = v[0:1] * v[0:1] + v[1:2] * v[1:2] + v[2:3] * v[2:3]
        nrm = jnp.sqrt(n2) + 1e-08
        pv = lax.dot_general(wp, v, (((1,), (0,)), ((), ())),
                             preferred_element_type=jnp.float32)
        planes = pv / nrm
        val = nrm * planes * jnp.abs(planes)
        acc = val if acc is None else jnp.maximum(acc, val)
    out = lax.dot_general(ws, acc, (((1,), (0,)), ((), ())),
                          preferred_element_type=jnp.float32)
    out_ref[0] = jnp.maximum(out + bs, 0.0)


def _tail_tc(grouped, xyz_t, w_planes, w_shapes, b_shapes_col):
    b_sz, _, _, m = grouped.shape
    np_, ns_ = w_planes.shape[0], w_shapes.shape[0]
    return pl.pallas_call(
        _tail_kernel,
        grid=(b_sz, m // _TILE_T),
        in_specs=[
            pl.BlockSpec((1, _K, 3, _TILE_T), lambda i, j: (i, 0, 0, j)),
            pl.BlockSpec((1, 3, _TILE_T), lambda i, j: (i, 0, j)),
            pl.BlockSpec(w_planes.shape, lambda i, j: (0, 0)),
            pl.BlockSpec(w_shapes.shape, lambda i, j: (0, 0)),
            pl.BlockSpec((ns_, 1), lambda i, j: (0, 0)),
        ],
        out_specs=pl.BlockSpec((1, ns_, _TILE_T), lambda i, j: (i, 0, j)),
        out_shape=jax.ShapeDtypeStruct((b_sz, ns_, m), jnp.float32),
    )(grouped, xyz_t, w_planes, w_shapes, b_shapes_col)


def kernel(xyz, W_planes, W_shapes, b_shapes):
    xyz_t = jnp.transpose(xyz, (0, 2, 1))
    idx = _knn_idx(xyz, xyz_t)
    grouped = _gather_sc(xyz_t, idx)
    shapes = _tail_tc(grouped, xyz_t, W_planes, W_shapes,
                      b_shapes.reshape(-1, 1))
    return (shapes, xyz, idx)
